# Initial kernel scaffold; baseline (speedup 1.0000x reference)
import jax
import jax.numpy as jnp
from jax.experimental import pallas as pl


def kernel(x, adj, W_heads, a_heads, W_out, a_out):
    raise NotImplementedError("write your pallas kernel here")


# speedup vs baseline: 1.1713x; 1.1713x over previous
import jax
import jax.numpy as jnp
from jax.experimental import pallas as pl

N = 2048
F_IN = 512
HID = 128
OUT = 256
HEADS = 8
ALPHA = 0.2
BM = 256

NEG = -9e15


def _leaky(x):
    return jnp.where(x >= 0, x, ALPHA * x)


def _elu(x):
    return jnp.where(x > 0, x, jnp.exp(x) - 1.0)


def _masked_softmax(e, mask):
    att = jnp.where(mask, e, NEG)
    m = jnp.max(att, axis=1, keepdims=True)
    p = jnp.exp(att - m)
    s = jnp.sum(p, axis=1, keepdims=True)
    return p / s


def _proj1_kernel(x_ref, w_ref, amat_ref, wh_ref, f_ref):
    wh = jnp.dot(x_ref[...], w_ref[...], preferred_element_type=jnp.float32)
    wh_ref[...] = wh
    f_ref[...] = jnp.dot(wh, amat_ref[...], preferred_element_type=jnp.float32)


def _attn1_kernel(wh_ref, adj_ref, f1_ref, f2t_ref, wout_ref, aomat_ref,
                  wh2_ref, fo_ref):
    i = pl.program_id(0)
    h = pl.program_id(1)
    whk = wh_ref[:, pl.ds(h * HID, HID)]
    f1 = f1_ref[pl.ds(i * BM, BM), :]
    onehot = (jax.lax.broadcasted_iota(jnp.int32, (HEADS, 1), 0) == h
              ).astype(jnp.float32)
    f1c = jnp.dot(f1, onehot, preferred_element_type=jnp.float32)
    f2r = f2t_ref[pl.ds(h, 1), :]
    e = _leaky(f1c + f2r)
    att = _masked_softmax(e, adj_ref[...].astype(jnp.float32) > 0)
    hp = jnp.dot(att, whk, preferred_element_type=jnp.float32)
    hh = _elu(hp)
    wout_h = wout_ref[pl.ds(h * HID, HID), :]
    contrib = jnp.dot(hh, wout_h, preferred_element_type=jnp.float32)

    @pl.when(h == 0)
    def _():
        wh2_ref[...] = contrib

    @pl.when(h > 0)
    def _():
        wh2_ref[...] += contrib

    @pl.when(h == HEADS - 1)
    def _():
        fo_ref[...] = jnp.dot(wh2_ref[...], aomat_ref[...],
                              preferred_element_type=jnp.float32)


def _attn2_kernel(wh2_ref, adj_ref, fo_ref, fot_ref, out_ref):
    i = pl.program_id(0)
    f1 = fo_ref[pl.ds(i * BM, BM), 0:1]
    f2r = fot_ref[1:2, :]
    e = _leaky(f1 + f2r)
    att = _masked_softmax(e, adj_ref[...].astype(jnp.float32) > 0)
    hp = jnp.dot(att, wh2_ref[...], preferred_element_type=jnp.float32)
    out_ref[...] = _elu(hp)


def kernel(x, adj, W_heads, a_heads, W_out, a_out):
    f32 = jnp.float32
    adj8 = (adj > 0).astype(jnp.int8)
    w_all = jnp.transpose(W_heads, (1, 0, 2)).reshape(F_IN, HEADS * HID)
    a1 = a_heads[:, :HID, 0]
    a2 = a_heads[:, HID:, 0]
    eye = jnp.eye(HEADS, dtype=f32)
    amat1 = (a1[:, :, None] * eye[:, None, :]).reshape(HEADS * HID, HEADS)
    amat2 = (a2[:, :, None] * eye[:, None, :]).reshape(HEADS * HID, HEADS)
    amat = jnp.concatenate([amat1, amat2], axis=1)
    aomat = jnp.concatenate([a_out[:OUT], a_out[OUT:]], axis=1)

    wh, f = pl.pallas_call(
        _proj1_kernel,
        grid=(N // BM,),
        in_specs=[
            pl.BlockSpec((BM, F_IN), lambda i: (i, 0)),
            pl.BlockSpec((F_IN, HEADS * HID), lambda i: (0, 0)),
            pl.BlockSpec((HEADS * HID, 2 * HEADS), lambda i: (0, 0)),
        ],
        out_specs=[
            pl.BlockSpec((BM, HEADS * HID), lambda i: (i, 0)),
            pl.BlockSpec((BM, 2 * HEADS), lambda i: (i, 0)),
        ],
        out_shape=[
            jax.ShapeDtypeStruct((N, HEADS * HID), f32),
            jax.ShapeDtypeStruct((N, 2 * HEADS), f32),
        ],
    )(x, w_all, amat)

    f1 = f[:, :HEADS]
    f2t = f[:, HEADS:].T

    wh2, fo = pl.pallas_call(
        _attn1_kernel,
        grid=(N // BM, HEADS),
        in_specs=[
            pl.BlockSpec((N, HEADS * HID), lambda i, h: (0, 0)),
            pl.BlockSpec((BM, N), lambda i, h: (i, 0)),
            pl.BlockSpec((N, HEADS), lambda i, h: (0, 0)),
            pl.BlockSpec((HEADS, N), lambda i, h: (0, 0)),
            pl.BlockSpec((HEADS * HID, OUT), lambda i, h: (0, 0)),
            pl.BlockSpec((OUT, 2), lambda i, h: (0, 0)),
        ],
        out_specs=[
            pl.BlockSpec((BM, OUT), lambda i, h: (i, 0)),
            pl.BlockSpec((BM, 2), lambda i, h: (i, 0)),
        ],
        out_shape=[
            jax.ShapeDtypeStruct((N, OUT), f32),
            jax.ShapeDtypeStruct((N, 2), f32),
        ],
    )(wh, adj8, f1, f2t, W_out, aomat)

    fot = fo.T

    out = pl.pallas_call(
        _attn2_kernel,
        grid=(N // BM,),
        in_specs=[
            pl.BlockSpec((N, OUT), lambda i: (0, 0)),
            pl.BlockSpec((BM, N), lambda i: (i, 0)),
            pl.BlockSpec((N, 2), lambda i: (0, 0)),
            pl.BlockSpec((2, N), lambda i: (0, 0)),
        ],
        out_specs=pl.BlockSpec((BM, OUT), lambda i: (i, 0)),
        out_shape=jax.ShapeDtypeStruct((N, OUT), f32),
    )(wh2, adj8, fo, fot)

    return out


# speedup vs baseline: 1.3681x; 1.1680x over previous
import jax
import jax.numpy as jnp
from jax.experimental import pallas as pl

N = 2048
F_IN = 512
HID = 128
OUT = 256
HEADS = 8
ALPHA = 0.2
BM = 256
EPS = 1e-30


def _leaky(x):
    return jnp.maximum(x, ALPHA * x)


def _elu(x):
    return jnp.where(x > 0, x, jnp.exp(x) - 1.0)


def _proj1_kernel(x_ref, w_ref, amat_ref, wh_ref, f_ref, csum_ref):
    i = pl.program_id(0)
    wh = jnp.dot(x_ref[...], w_ref[...], preferred_element_type=jnp.float32)
    wh_ref[...] = wh
    f_ref[...] = jnp.dot(wh, amat_ref[...], preferred_element_type=jnp.float32)
    part = jnp.sum(wh, axis=0, keepdims=True)

    @pl.when(i == 0)
    def _():
        csum_ref[...] = part

    @pl.when(i > 0)
    def _():
        csum_ref[...] += part


def _attn1_kernel(wh_ref, adj_ref, f1_ref, f2t_ref, wout_ref, aomat_ref,
                  csum_ref, wh2_ref, fo_ref, csum2_ref):
    i = pl.program_id(0)
    h = pl.program_id(1)
    whk = wh_ref[:, pl.ds(h * HID, HID)]
    f1 = f1_ref[pl.ds(i * BM, BM), :]
    onehot = (jax.lax.broadcasted_iota(jnp.int32, (HEADS, 1), 0) == h
              ).astype(jnp.float32)
    f1c = jnp.dot(f1, onehot, preferred_element_type=jnp.float32)
    f2r = f2t_ref[pl.ds(h, 1), :]
    maskf = adj_ref[...].astype(jnp.float32)
    p = jnp.exp(_leaky(f1c + f2r)) * maskf
    s = jnp.sum(p, axis=1, keepdims=True) + (N * EPS)
    csum_h = csum_ref[:, pl.ds(h * HID, HID)]
    num = jnp.dot(p, whk, preferred_element_type=jnp.float32) + EPS * csum_h
    hh = _elu(num * (1.0 / s))
    wout_h = wout_ref[pl.ds(h * HID, HID), :]
    contrib = jnp.dot(hh, wout_h, preferred_element_type=jnp.float32)

    @pl.when(h == 0)
    def _():
        wh2_ref[...] = contrib

    @pl.when(h > 0)
    def _():
        wh2_ref[...] += contrib

    @pl.when(h == HEADS - 1)
    def _():
        wh2 = wh2_ref[...]
        fo_ref[...] = jnp.dot(wh2, aomat_ref[...],
                              preferred_element_type=jnp.float32)
        part2 = jnp.sum(wh2, axis=0, keepdims=True)

        @pl.when(i == 0)
        def _():
            csum2_ref[...] = part2

        @pl.when(i > 0)
        def _():
            csum2_ref[...] += part2


def _attn2_kernel(wh2_ref, adj_ref, fo_ref, fot_ref, csum2_ref, out_ref):
    i = pl.program_id(0)
    f1 = fo_ref[pl.ds(i * BM, BM), 0:1]
    f2r = fot_ref[1:2, :]
    maskf = adj_ref[...].astype(jnp.float32)
    p = jnp.exp(_leaky(f1 + f2r)) * maskf
    s = jnp.sum(p, axis=1, keepdims=True) + (N * EPS)
    num = (jnp.dot(p, wh2_ref[...], preferred_element_type=jnp.float32)
           + EPS * csum2_ref[...])
    out_ref[...] = _elu(num * (1.0 / s))


def kernel(x, adj, W_heads, a_heads, W_out, a_out):
    f32 = jnp.float32
    adj8 = (adj > 0).astype(jnp.int8)
    w_all = jnp.transpose(W_heads, (1, 0, 2)).reshape(F_IN, HEADS * HID)
    a1 = a_heads[:, :HID, 0]
    a2 = a_heads[:, HID:, 0]
    eye = jnp.eye(HEADS, dtype=f32)
    amat1 = (a1[:, :, None] * eye[:, None, :]).reshape(HEADS * HID, HEADS)
    amat2 = (a2[:, :, None] * eye[:, None, :]).reshape(HEADS * HID, HEADS)
    amat = jnp.concatenate([amat1, amat2], axis=1)
    aomat = jnp.concatenate([a_out[:OUT], a_out[OUT:]], axis=1)

    wh, f, csum = pl.pallas_call(
        _proj1_kernel,
        grid=(N // BM,),
        in_specs=[
            pl.BlockSpec((BM, F_IN), lambda i: (i, 0)),
            pl.BlockSpec((F_IN, HEADS * HID), lambda i: (0, 0)),
            pl.BlockSpec((HEADS * HID, 2 * HEADS), lambda i: (0, 0)),
        ],
        out_specs=[
            pl.BlockSpec((BM, HEADS * HID), lambda i: (i, 0)),
            pl.BlockSpec((BM, 2 * HEADS), lambda i: (i, 0)),
            pl.BlockSpec((1, HEADS * HID), lambda i: (0, 0)),
        ],
        out_shape=[
            jax.ShapeDtypeStruct((N, HEADS * HID), f32),
            jax.ShapeDtypeStruct((N, 2 * HEADS), f32),
            jax.ShapeDtypeStruct((1, HEADS * HID), f32),
        ],
    )(x, w_all, amat)

    f1 = f[:, :HEADS]
    f2t = f[:, HEADS:].T

    wh2, fo, csum2 = pl.pallas_call(
        _attn1_kernel,
        grid=(N // BM, HEADS),
        in_specs=[
            pl.BlockSpec((N, HEADS * HID), lambda i, h: (0, 0)),
            pl.BlockSpec((BM, N), lambda i, h: (i, 0)),
            pl.BlockSpec((N, HEADS), lambda i, h: (0, 0)),
            pl.BlockSpec((HEADS, N), lambda i, h: (0, 0)),
            pl.BlockSpec((HEADS * HID, OUT), lambda i, h: (0, 0)),
            pl.BlockSpec((OUT, 2), lambda i, h: (0, 0)),
            pl.BlockSpec((1, HEADS * HID), lambda i, h: (0, 0)),
        ],
        out_specs=[
            pl.BlockSpec((BM, OUT), lambda i, h: (i, 0)),
            pl.BlockSpec((BM, 2), lambda i, h: (i, 0)),
            pl.BlockSpec((1, OUT), lambda i, h: (0, 0)),
        ],
        out_shape=[
            jax.ShapeDtypeStruct((N, OUT), f32),
            jax.ShapeDtypeStruct((N, 2), f32),
            jax.ShapeDtypeStruct((1, OUT), f32),
        ],
    )(wh, adj8, f1, f2t, W_out, aomat, csum)

    fot = fo.T

    out = pl.pallas_call(
        _attn2_kernel,
        grid=(N // BM,),
        in_specs=[
            pl.BlockSpec((N, OUT), lambda i: (0, 0)),
            pl.BlockSpec((BM, N), lambda i: (i, 0)),
            pl.BlockSpec((N, 2), lambda i: (0, 0)),
            pl.BlockSpec((2, N), lambda i: (0, 0)),
            pl.BlockSpec((1, OUT), lambda i: (0, 0)),
        ],
        out_specs=pl.BlockSpec((BM, OUT), lambda i: (i, 0)),
        out_shape=jax.ShapeDtypeStruct((N, OUT), f32),
    )(wh2, adj8, fo, fot, csum2)

    return out


# speedup vs baseline: 1.7540x; 1.2821x over previous
import math

import jax
import jax.numpy as jnp
from jax.experimental import pallas as pl

N = 2048
F_IN = 512
HID = 128
OUT = 256
HEADS = 8
ALPHA = 0.2
BM = 512
BP = 256
EPS = 1e-30
LOG2E = math.log2(math.e)
MASK_BIAS = -1e6


def _leaky(x):
    return jnp.maximum(x, ALPHA * x)


def _elu(x):
    return jnp.where(x > 0, x, jnp.exp(x) - 1.0)


def _proj1_kernel(x_ref, w_ref, amat_ref, adj_ref, wh_ref, f_ref, csum_ref,
                  bias_ref):
    i = pl.program_id(0)
    wh = jnp.dot(x_ref[...], w_ref[...], preferred_element_type=jnp.float32)
    wh_ref[...] = wh
    f_ref[...] = jnp.dot(wh, amat_ref[...], preferred_element_type=jnp.float32)
    bias_ref[...] = jnp.where(adj_ref[...] > 0, 0.0, MASK_BIAS)
    part = jnp.sum(wh, axis=0, keepdims=True)

    @pl.when(i == 0)
    def _():
        csum_ref[...] = part

    @pl.when(i > 0)
    def _():
        csum_ref[...] += part


def _attn1_kernel(wh_ref, bias_ref, f1_ref, f2t_ref, wout_ref, aomat_ref,
                  csum_ref, wh2_ref, fo_ref, csum2_ref):
    i = pl.program_id(0)
    h = pl.program_id(1)
    whk = wh_ref[:, pl.ds(h * HID, HID)]
    f1 = f1_ref[pl.ds(i * BM, BM), :]
    onehot = (jax.lax.broadcasted_iota(jnp.int32, (HEADS, 1), 0) == h
              ).astype(jnp.float32)
    f1c = jnp.dot(f1, onehot, preferred_element_type=jnp.float32)
    f2r = f2t_ref[pl.ds(h, 1), :]
    p = jnp.exp2(_leaky(f1c + f2r) + bias_ref[...])
    s = jnp.sum(p, axis=1, keepdims=True) + (N * EPS)
    csum_h = csum_ref[:, pl.ds(h * HID, HID)]
    num = jnp.dot(p, whk, preferred_element_type=jnp.float32) + EPS * csum_h
    hh = _elu(num * (1.0 / s))
    wout_h = wout_ref[pl.ds(h * HID, HID), :]
    contrib = jnp.dot(hh, wout_h, preferred_element_type=jnp.float32)

    @pl.when(h == 0)
    def _():
        wh2_ref[...] = contrib

    @pl.when(h > 0)
    def _():
        wh2_ref[...] += contrib

    @pl.when(h == HEADS - 1)
    def _():
        wh2 = wh2_ref[...]
        fo_ref[...] = jnp.dot(wh2, aomat_ref[...],
                              preferred_element_type=jnp.float32)
        part2 = jnp.sum(wh2, axis=0, keepdims=True)

        @pl.when(i == 0)
        def _():
            csum2_ref[...] = part2

        @pl.when(i > 0)
        def _():
            csum2_ref[...] += part2


def _attn2_kernel(wh2_ref, bias_ref, fo_ref, fot_ref, csum2_ref, out_ref):
    i = pl.program_id(0)
    f1 = fo_ref[pl.ds(i * BM, BM), 0:1]
    f2r = fot_ref[1:2, :]
    p = jnp.exp2(_leaky(f1 + f2r) + bias_ref[...])
    s = jnp.sum(p, axis=1, keepdims=True) + (N * EPS)
    num = (jnp.dot(p, wh2_ref[...], preferred_element_type=jnp.float32)
           + EPS * csum2_ref[...])
    out_ref[...] = _elu(num * (1.0 / s))


def kernel(x, adj, W_heads, a_heads, W_out, a_out):
    f32 = jnp.float32
    w_all = jnp.transpose(W_heads, (1, 0, 2)).reshape(F_IN, HEADS * HID)
    a1 = a_heads[:, :HID, 0]
    a2 = a_heads[:, HID:, 0]
    eye = jnp.eye(HEADS, dtype=f32)
    amat1 = (a1[:, :, None] * eye[:, None, :]).reshape(HEADS * HID, HEADS)
    amat2 = (a2[:, :, None] * eye[:, None, :]).reshape(HEADS * HID, HEADS)
    amat = jnp.concatenate([amat1, amat2], axis=1) * LOG2E
    aomat = jnp.concatenate([a_out[:OUT], a_out[OUT:]], axis=1) * LOG2E

    wh, f, csum, bias = pl.pallas_call(
        _proj1_kernel,
        grid=(N // BP,),
        in_specs=[
            pl.BlockSpec((BP, F_IN), lambda i: (i, 0)),
            pl.BlockSpec((F_IN, HEADS * HID), lambda i: (0, 0)),
            pl.BlockSpec((HEADS * HID, 2 * HEADS), lambda i: (0, 0)),
            pl.BlockSpec((BP, N), lambda i: (i, 0)),
        ],
        out_specs=[
            pl.BlockSpec((BP, HEADS * HID), lambda i: (i, 0)),
            pl.BlockSpec((BP, 2 * HEADS), lambda i: (i, 0)),
            pl.BlockSpec((1, HEADS * HID), lambda i: (0, 0)),
            pl.BlockSpec((BP, N), lambda i: (i, 0)),
        ],
        out_shape=[
            jax.ShapeDtypeStruct((N, HEADS * HID), f32),
            jax.ShapeDtypeStruct((N, 2 * HEADS), f32),
            jax.ShapeDtypeStruct((1, HEADS * HID), f32),
            jax.ShapeDtypeStruct((N, N), f32),
        ],
    )(x, w_all, amat, adj)

    f1 = f[:, :HEADS]
    f2t = f[:, HEADS:].T

    wh2, fo, csum2 = pl.pallas_call(
        _attn1_kernel,
        grid=(N // BM, HEADS),
        in_specs=[
            pl.BlockSpec((N, HEADS * HID), lambda i, h: (0, 0)),
            pl.BlockSpec((BM, N), lambda i, h: (i, 0)),
            pl.BlockSpec((N, HEADS), lambda i, h: (0, 0)),
            pl.BlockSpec((HEADS, N), lambda i, h: (0, 0)),
            pl.BlockSpec((HEADS * HID, OUT), lambda i, h: (0, 0)),
            pl.BlockSpec((OUT, 2), lambda i, h: (0, 0)),
            pl.BlockSpec((1, HEADS * HID), lambda i, h: (0, 0)),
        ],
        out_specs=[
            pl.BlockSpec((BM, OUT), lambda i, h: (i, 0)),
            pl.BlockSpec((BM, 2), lambda i, h: (i, 0)),
            pl.BlockSpec((1, OUT), lambda i, h: (0, 0)),
        ],
        out_shape=[
            jax.ShapeDtypeStruct((N, OUT), f32),
            jax.ShapeDtypeStruct((N, 2), f32),
            jax.ShapeDtypeStruct((1, OUT), f32),
        ],
    )(wh, bias, f1, f2t, W_out, aomat, csum)

    fot = fo.T

    out = pl.pallas_call(
        _attn2_kernel,
        grid=(N // BM,),
        in_specs=[
            pl.BlockSpec((N, OUT), lambda i: (0, 0)),
            pl.BlockSpec((BM, N), lambda i: (i, 0)),
            pl.BlockSpec((N, 2), lambda i: (0, 0)),
            pl.BlockSpec((2, N), lambda i: (0, 0)),
            pl.BlockSpec((1, OUT), lambda i: (0, 0)),
        ],
        out_specs=pl.BlockSpec((BM, OUT), lambda i: (i, 0)),
        out_shape=jax.ShapeDtypeStruct((N, OUT), f32),
    )(wh2, bias, fo, fot, csum2)

    return out


# speedup vs baseline: 1.8505x; 1.0550x over previous
import math

import jax
import jax.numpy as jnp
from jax.experimental import pallas as pl

N = 2048
F_IN = 512
HID = 128
OUT = 256
HEADS = 8
ALPHA = 0.2
BM = 512
BP = 256
CK = 512
EPS = 1e-30
LOG2E = math.log2(math.e)
MASK_BIAS = -1e6


def _leaky(x):
    return jnp.maximum(x, ALPHA * x)


def _elu(x):
    return jnp.where(x > 0, x, jnp.exp(x) - 1.0)


def _proj1_kernel(x_ref, w_ref, amat_ref, adj_ref, wh_ref, f_ref, csum_ref,
                  bias_ref):
    i = pl.program_id(0)
    wh = jnp.dot(x_ref[...], w_ref[...], preferred_element_type=jnp.float32)
    wh_ref[...] = wh
    f_ref[...] = jnp.dot(wh, amat_ref[...], preferred_element_type=jnp.float32)
    bias_ref[...] = jnp.where(adj_ref[...] > 0, 0.0, MASK_BIAS)
    part = jnp.sum(wh, axis=0, keepdims=True)

    @pl.when(i == 0)
    def _():
        csum_ref[...] = part

    @pl.when(i > 0)
    def _():
        csum_ref[...] += part


def _attn1_kernel(wh_ref, bias_ref, f1_ref, f2t_ref, wout_ref, aomat_ref,
                  csum_ref, wh2_ref, fo_ref, csum2_ref):
    i = pl.program_id(0)
    h = pl.program_id(1)
    f1 = f1_ref[pl.ds(i * BM, BM), :]
    onehot = (jax.lax.broadcasted_iota(jnp.int32, (HEADS, 1), 0) == h
              ).astype(jnp.float32)
    f1c = jnp.dot(f1, onehot, preferred_element_type=jnp.float32)

    num = jnp.zeros((BM, HID), jnp.float32)
    s = jnp.zeros((BM, 1), jnp.float32)
    for c in range(N // CK):
        f2c = f2t_ref[pl.ds(h, 1), c * CK:(c + 1) * CK]
        biasc = bias_ref[:, c * CK:(c + 1) * CK]
        pc = jnp.exp2(_leaky(f1c + f2c) + biasc)
        s = s + jnp.sum(pc, axis=1, keepdims=True)
        whc = wh_ref[pl.ds(c * CK, CK), pl.ds(h * HID, HID)]
        num = num + jnp.dot(pc, whc, preferred_element_type=jnp.float32)

    s = s + (N * EPS)
    csum_h = csum_ref[:, pl.ds(h * HID, HID)]
    hh = _elu((num + EPS * csum_h) * (1.0 / s))
    wout_h = wout_ref[pl.ds(h * HID, HID), :]
    contrib = jnp.dot(hh, wout_h, preferred_element_type=jnp.float32)

    @pl.when(h == 0)
    def _():
        wh2_ref[...] = contrib

    @pl.when(h > 0)
    def _():
        wh2_ref[...] += contrib

    @pl.when(h == HEADS - 1)
    def _():
        wh2 = wh2_ref[...]
        fo_ref[...] = jnp.dot(wh2, aomat_ref[...],
                              preferred_element_type=jnp.float32)
        part2 = jnp.sum(wh2, axis=0, keepdims=True)

        @pl.when(i == 0)
        def _():
            csum2_ref[...] = part2

        @pl.when(i > 0)
        def _():
            csum2_ref[...] += part2


def _attn2_kernel(wh2_ref, bias_ref, fo_ref, fot_ref, csum2_ref, out_ref):
    i = pl.program_id(0)
    f1 = fo_ref[pl.ds(i * BM, BM), 0:1]

    num = jnp.zeros((BM, OUT), jnp.float32)
    s = jnp.zeros((BM, 1), jnp.float32)
    for c in range(N // CK):
        f2c = fot_ref[1:2, c * CK:(c + 1) * CK]
        biasc = bias_ref[:, c * CK:(c + 1) * CK]
        pc = jnp.exp2(_leaky(f1 + f2c) + biasc)
        s = s + jnp.sum(pc, axis=1, keepdims=True)
        wh2c = wh2_ref[c * CK:(c + 1) * CK, :]
        num = num + jnp.dot(pc, wh2c, preferred_element_type=jnp.float32)

    s = s + (N * EPS)
    out_ref[...] = _elu((num + EPS * csum2_ref[...]) * (1.0 / s))


def kernel(x, adj, W_heads, a_heads, W_out, a_out):
    f32 = jnp.float32
    w_all = jnp.transpose(W_heads, (1, 0, 2)).reshape(F_IN, HEADS * HID)
    a1 = a_heads[:, :HID, 0]
    a2 = a_heads[:, HID:, 0]
    eye = jnp.eye(HEADS, dtype=f32)
    amat1 = (a1[:, :, None] * eye[:, None, :]).reshape(HEADS * HID, HEADS)
    amat2 = (a2[:, :, None] * eye[:, None, :]).reshape(HEADS * HID, HEADS)
    amat = jnp.concatenate([amat1, amat2], axis=1) * LOG2E
    aomat = jnp.concatenate([a_out[:OUT], a_out[OUT:]], axis=1) * LOG2E

    wh, f, csum, bias = pl.pallas_call(
        _proj1_kernel,
        grid=(N // BP,),
        in_specs=[
            pl.BlockSpec((BP, F_IN), lambda i: (i, 0)),
            pl.BlockSpec((F_IN, HEADS * HID), lambda i: (0, 0)),
            pl.BlockSpec((HEADS * HID, 2 * HEADS), lambda i: (0, 0)),
            pl.BlockSpec((BP, N), lambda i: (i, 0)),
        ],
        out_specs=[
            pl.BlockSpec((BP, HEADS * HID), lambda i: (i, 0)),
            pl.BlockSpec((BP, 2 * HEADS), lambda i: (i, 0)),
            pl.BlockSpec((1, HEADS * HID), lambda i: (0, 0)),
            pl.BlockSpec((BP, N), lambda i: (i, 0)),
        ],
        out_shape=[
            jax.ShapeDtypeStruct((N, HEADS * HID), f32),
            jax.ShapeDtypeStruct((N, 2 * HEADS), f32),
            jax.ShapeDtypeStruct((1, HEADS * HID), f32),
            jax.ShapeDtypeStruct((N, N), f32),
        ],
    )(x, w_all, amat, adj)

    f1 = f[:, :HEADS]
    f2t = f[:, HEADS:].T

    wh2, fo, csum2 = pl.pallas_call(
        _attn1_kernel,
        grid=(N // BM, HEADS),
        in_specs=[
            pl.BlockSpec((N, HEADS * HID), lambda i, h: (0, 0)),
            pl.BlockSpec((BM, N), lambda i, h: (i, 0)),
            pl.BlockSpec((N, HEADS), lambda i, h: (0, 0)),
            pl.BlockSpec((HEADS, N), lambda i, h: (0, 0)),
            pl.BlockSpec((HEADS * HID, OUT), lambda i, h: (0, 0)),
            pl.BlockSpec((OUT, 2), lambda i, h: (0, 0)),
            pl.BlockSpec((1, HEADS * HID), lambda i, h: (0, 0)),
        ],
        out_specs=[
            pl.BlockSpec((BM, OUT), lambda i, h: (i, 0)),
            pl.BlockSpec((BM, 2), lambda i, h: (i, 0)),
            pl.BlockSpec((1, OUT), lambda i, h: (0, 0)),
        ],
        out_shape=[
            jax.ShapeDtypeStruct((N, OUT), f32),
            jax.ShapeDtypeStruct((N, 2), f32),
            jax.ShapeDtypeStruct((1, OUT), f32),
        ],
    )(wh, bias, f1, f2t, W_out, aomat, csum)

    fot = fo.T

    out = pl.pallas_call(
        _attn2_kernel,
        grid=(N // BM,),
        in_specs=[
            pl.BlockSpec((N, OUT), lambda i: (0, 0)),
            pl.BlockSpec((BM, N), lambda i: (i, 0)),
            pl.BlockSpec((N, 2), lambda i: (0, 0)),
            pl.BlockSpec((2, N), lambda i: (0, 0)),
            pl.BlockSpec((1, OUT), lambda i: (0, 0)),
        ],
        out_specs=pl.BlockSpec((BM, OUT), lambda i: (i, 0)),
        out_shape=jax.ShapeDtypeStruct((N, OUT), f32),
    )(wh2, bias, fo, fot, csum2)

    return out
